# baseline (speedup 1.0000x reference)
import functools

import jax
import jax.numpy as jnp
from jax import lax
from jax.experimental import pallas as pl
from jax.experimental.pallas import tpu as pltpu
from jax.experimental.pallas import tpu_sc as plsc

VOCAB = 1000000
DIM = 64
B = 4096
L = 200

NC = 2
NS = 16
NW = NC * NS
LANES = 16

TOTAL = B * L
CHUNK = 128
PER_W = TOTAL // NW
STEPS = PER_W // CHUNK
NBUF = 2
GROUPS = STEPS // NBUF


def _sc_gather(table2, idx2d):
    mesh = plsc.VectorSubcoreMesh(core_axis_name="c", subcore_axis_name="s")

    @functools.partial(
        pl.kernel,
        mesh=mesh,
        out_type=jax.ShapeDtypeStruct((TOTAL, DIM), jnp.float32),
        compiler_params=pltpu.CompilerParams(needs_layout_passes=False),
        scratch_types=[
            pltpu.VMEM((STEPS, CHUNK), jnp.int32),
        ]
        + [pltpu.VMEM((CHUNK,), jnp.int32) for _ in range(NBUF)]
        + [pltpu.VMEM((CHUNK, 2 * DIM), jnp.float32) for _ in range(NBUF)]
        + [pltpu.VMEM((CHUNK, DIM), jnp.float32) for _ in range(NBUF)]
        + [pltpu.SemaphoreType.DMA for _ in range(2 * NBUF)],
    )
    def k(table_hbm, idx_hbm, out_hbm, idx_v, *scratch):
        qb = list(scratch[:NBUF])
        gb = list(scratch[NBUF : 2 * NBUF])
        ob = list(scratch[2 * NBUF : 3 * NBUF])
        gsems = list(scratch[3 * NBUF : 4 * NBUF])
        wsems = list(scratch[4 * NBUF : 5 * NBUF])

        wid = lax.axis_index("s") * NC + lax.axis_index("c")
        base = wid * PER_W

        pltpu.sync_copy(idx_hbm.at[pl.ds(wid * STEPS, STEPS)], idx_v)

        iota = lax.iota(jnp.int32, LANES)

        def g_start(j, b):
            for t in range(CHUNK // LANES):
                v = idx_v[j, pl.ds(t * LANES, LANES)]
                qb[b][pl.ds(t * LANES, LANES)] = lax.shift_right_logical(v, 1)
            pltpu.async_copy(table_hbm.at[qb[b]], gb[b], gsems[b])

        def g_wait(b):
            pltpu.make_async_copy(table_hbm.at[qb[b]], gb[b], gsems[b]).wait()

        def select(j, b):
            def grp(t, carry):
                v = idx_v[j, pl.ds(t * LANES, LANES)]
                colbase = (v & 1) << 6
                row = t * LANES + iota
                for c in range(DIM):
                    vals = plsc.load_gather(gb[b], [row, colbase + c])
                    plsc.store_scatter(
                        ob[b], [row, jnp.full((LANES,), c, jnp.int32)], vals
                    )
                return carry

            lax.fori_loop(0, CHUNK // LANES, grp, 0)

        def w_start(j, b):
            pltpu.async_copy(ob[b], out_hbm.at[pl.ds(base + j * CHUNK, CHUNK)], wsems[b])

        def w_wait(j, b):
            pltpu.make_async_copy(
                ob[b], out_hbm.at[pl.ds(base + j * CHUNK, CHUNK)], wsems[b]
            ).wait()

        for b in range(NBUF):
            g_start(b, b)

        def group(g, carry):
            j0 = g * NBUF
            for b in range(NBUF):
                g_wait(b)
                select(j0 + b, b)
                w_start(j0 + b, b)
            for b in range(NBUF):
                w_wait(j0 + b, b)
                g_start(j0 + NBUF + b, b)
            return carry

        lax.fori_loop(0, GROUPS - 1, group, 0)

        j0 = (GROUPS - 1) * NBUF
        for b in range(NBUF):
            g_wait(b)
            select(j0 + b, b)
            w_start(j0 + b, b)
        for b in range(NBUF):
            w_wait(j0 + b, b)

    return k(table2, idx2d)


def kernel(X_word, X_char, word_embed):
    del X_char
    idx2d = X_word.reshape(TOTAL // CHUNK, CHUNK)
    table2 = word_embed.reshape(VOCAB // 2, 2 * DIM)
    flat = _sc_gather(table2, idx2d)
    return flat.reshape(B, L, DIM)


# speedup vs baseline: 3.0030x; 3.0030x over previous
import functools

import jax
import jax.numpy as jnp
from jax import lax
from jax.experimental import pallas as pl
from jax.experimental.pallas import tpu as pltpu
from jax.experimental.pallas import tpu_sc as plsc

VOCAB = 1000000
DIM = 64
B = 4096
L = 200

NC = 2
NS = 16
NW = NC * NS

TOTAL = B * L
CHUNK = 128
PER_W = TOTAL // NW
STEPS = PER_W // CHUNK
NBUF = 4
GROUPS = STEPS // NBUF


def _sc_gather(table, idx2d):
    mesh = plsc.VectorSubcoreMesh(core_axis_name="c", subcore_axis_name="s")

    @functools.partial(
        pl.kernel,
        mesh=mesh,
        out_type=jax.ShapeDtypeStruct((TOTAL, 2 * DIM), jnp.float32),
        compiler_params=pltpu.CompilerParams(use_tc_tiling_on_sc=False),
        scratch_types=[
            pltpu.VMEM((STEPS, CHUNK), jnp.int32),
        ]
        + [pltpu.VMEM((CHUNK, DIM), jnp.float32) for _ in range(NBUF)]
        + [pltpu.SemaphoreType.DMA for _ in range(2 * NBUF)],
    )
    def k(table_hbm, idx_hbm, out_hbm, idx_v, *bufs_and_sems):
        rows = list(bufs_and_sems[:NBUF])
        gsems = list(bufs_and_sems[NBUF : 2 * NBUF])
        wsems = list(bufs_and_sems[2 * NBUF : 3 * NBUF])

        wid = lax.axis_index("s") * NC + lax.axis_index("c")
        base = wid * PER_W

        pltpu.sync_copy(idx_hbm.at[pl.ds(wid * STEPS, STEPS)], idx_v)

        def g_start(j, b):
            pltpu.async_copy(table_hbm.at[idx_v.at[j]], rows[b], gsems[b])

        def g_wait(j, b):
            pltpu.make_async_copy(table_hbm.at[idx_v.at[j]], rows[b], gsems[b]).wait()

        def w_start(j, b):
            pltpu.async_copy(
                rows[b],
                out_hbm.at[pl.ds(base + j * CHUNK, CHUNK), pl.ds(0, DIM)],
                wsems[b],
            )

        def w_wait(j, b):
            pltpu.make_async_copy(
                rows[b],
                out_hbm.at[pl.ds(base + j * CHUNK, CHUNK), pl.ds(0, DIM)],
                wsems[b],
            ).wait()

        for b in range(NBUF):
            g_start(b, b)

        def group(g, carry):
            j0 = g * NBUF
            for b in range(NBUF):
                g_wait(j0 + b, b)
                w_start(j0 + b, b)
            for b in range(NBUF):
                w_wait(j0 + b, b)
                g_start(j0 + NBUF + b, b)
            return carry

        lax.fori_loop(0, GROUPS - 1, group, 0)

        j0 = (GROUPS - 1) * NBUF
        for b in range(NBUF):
            g_wait(j0 + b, b)
            w_start(j0 + b, b)
        for b in range(NBUF):
            w_wait(j0 + b, b)

    return k(table, idx2d)


def kernel(X_word, X_char, word_embed):
    del X_char
    idx2d = X_word.reshape(TOTAL // CHUNK, CHUNK)
    padded = _sc_gather(word_embed, idx2d)
    return padded[:, :DIM].reshape(B, L, DIM)
